# baseline (speedup 1.0000x reference)
import functools

import jax
import jax.numpy as jnp
import numpy as np
from jax import lax
from jax.experimental import pallas as pl
from jax.experimental.pallas import tpu as pltpu
from jax.experimental.pallas import tpu_sc as plsc

NC = 2
NS = 16
CHUNK = 80
PACK = 16384


def _make_sc_segsum(n, e, w):
    nw = NC * NS
    per_w = e // nw
    assert per_w * nw == e and per_w % CHUNK == 0
    nch = per_w // CHUNK
    assert nch >= 8 and nch % 3 == 2
    rpt = -(-n // (8 * NS)) * 8
    n_pad = rpt * NS
    mesh = plsc.VectorSubcoreMesh(core_axis_name="c", subcore_axis_name="s")

    def body(table, pk, zeros, out, pk_v,
             sbuf0, sbuf1, sbuf2, dbuf0, dbuf1, dbuf2,
             rows0, rows1, rows2, acc, gsem, ssem):
        c = lax.axis_index("c")
        s = lax.axis_index("s")
        g = s * NC + c
        pltpu.sync_copy(zeros, acc.at[pl.ds(s * rpt, rpt)])
        pltpu.sync_copy(pk.at[g], pk_v)
        plsc.subcore_barrier()

        dummy = table.at[pl.ds(0, CHUNK)]
        sbufs = (sbuf0, sbuf1, sbuf2)
        dbufs = (dbuf0, dbuf1, dbuf2)
        rows = (rows0, rows1, rows2)

        def unpack(j, slot):
            base = j * CHUNK
            for k in range(CHUNK // 16):
                p16 = pk_v[pl.ds(base + k * 16, 16)]
                sbufs[slot][pl.ds(k * 16, 16)] = lax.bitwise_and(
                    p16, PACK - 1)
                dbufs[slot][pl.ds(k * 16, 16)] = lax.shift_right_logical(
                    p16, 14)

        def gather(slot):
            pltpu.async_copy(table.at[sbufs[slot]], rows[slot], gsem)

        def scatter(slot):
            pltpu.async_copy(rows[slot], acc.at[dbufs[slot]], ssem,
                             add=True)

        def drain(slot, sem):
            pltpu.make_async_copy(dummy, rows[slot], sem).wait()

        unpack(0, 0)
        gather(0)
        unpack(1, 1)
        gather(1)
        drain(0, gsem)
        scatter(0)
        unpack(2, 2)
        gather(2)
        drain(1, gsem)
        scatter(1)

        def triple(t, carry):
            for k in range(3):
                j = 3 * t + k
                drain(k, ssem)
                unpack(j, k)
                gather(k)
                drain((k + 2) % 3, gsem)
                scatter((k + 2) % 3)
            return carry

        lax.fori_loop(1, (nch - 2) // 3, triple, 0)
        drain(0, ssem)
        unpack(nch - 2, 0)
        gather(0)
        drain(2, gsem)
        scatter(2)
        drain(1, ssem)
        unpack(nch - 1, 1)
        gather(1)
        drain(0, gsem)
        scatter(0)
        drain(1, gsem)
        scatter(1)
        drain(2, ssem)
        drain(0, ssem)
        drain(1, ssem)
        plsc.subcore_barrier()
        pltpu.sync_copy(acc.at[pl.ds(s * rpt, rpt)],
                        out.at[pl.ds(c * n_pad + s * rpt, rpt)])

    out_type = [jax.ShapeDtypeStruct((NC * n_pad, w), jnp.float32)]
    scratch = (
        [pltpu.VMEM((per_w,), jnp.int32)]
        + [pltpu.VMEM((CHUNK,), jnp.int32)] * 6
        + [pltpu.VMEM((CHUNK, w), jnp.float32)] * 3
        + [
            pltpu.VMEM_SHARED((n_pad, w), jnp.float32),
            pltpu.SemaphoreType.DMA,
            pltpu.SemaphoreType.DMA,
        ]
    )
    return pl.kernel(
        body, out_type=out_type, mesh=mesh, scratch_types=scratch,
        compiler_params=pltpu.CompilerParams(
            needs_layout_passes=False, use_tc_tiling_on_sc=False),
    ), n_pad


def _dotT(a, b):
    return lax.dot_general(a, b, (((1,), (1,)), ((), ())),
                           preferred_element_type=jnp.float32)


def _layer1_body(pa_ref, pb_ref, xp_ref, wl_ref, wr_ref, b_ref,
                 h_ref, rcp_ref):
    p = pa_ref[...] + pb_ref[...]
    d = wl_ref.shape[1]
    cnt = p[:, d:d + 1]
    rcp = 1.0 / jnp.maximum(cnt, 1.0)
    mean = p[:, :d] * rcp
    h = (_dotT(mean, wl_ref[...]) + _dotT(xp_ref[:, :d], wr_ref[...])
         + b_ref[...])
    h_ref[...] = jnp.maximum(h, 0.0)
    rcp_ref[...] = rcp


def _layer2_body(pa_ref, pb_ref, h_ref, rcp_ref, wl_ref, wr_ref, b_ref,
                 wlin_ref, blin_ref, o_ref):
    mean = (pa_ref[...] + pb_ref[...]) * rcp_ref[...]
    z = _dotT(mean, wl_ref[...]) + _dotT(h_ref[...], wr_ref[...]) + b_ref[...]
    z = jnp.maximum(z, 0.0)
    o_ref[...] = (jnp.sum(z * wlin_ref[...], axis=1, keepdims=True)
                  + blin_ref[0, 0])


def kernel(x, edge_index, W1l, W1r, b1, W2l, W2r, b2, Wlin, blin):
    n, d = x.shape
    e = edge_index.shape[1]
    h_dim = W1l.shape[0]
    per_w = e // (NC * NS)
    pk = (edge_index[0] + edge_index[1] * PACK).reshape(NC * NS, per_w)

    w1 = d + 8
    seg1, n_pad = _make_sc_segsum(n, e, w1)
    xp = jnp.zeros((n_pad, w1), jnp.float32)
    xp = lax.dynamic_update_slice(
        xp,
        jnp.concatenate([x, jnp.ones((n, 1), jnp.float32)], axis=1),
        (0, 0))

    zeros1 = jnp.zeros((n_pad // NS, w1), jnp.float32)
    (p1,) = seg1(xp, pk, zeros1)

    rb = n_pad // 8
    assert rb % 8 == 0
    grid = n_pad // rb
    h, rcp = pl.pallas_call(
        _layer1_body,
        grid=(grid,),
        in_specs=[
            pl.BlockSpec((rb, w1), lambda i: (i, 0)),
            pl.BlockSpec((rb, w1), lambda i: (i + 8, 0)),
            pl.BlockSpec((rb, w1), lambda i: (i, 0)),
            pl.BlockSpec((h_dim, d), lambda i: (0, 0)),
            pl.BlockSpec((h_dim, d), lambda i: (0, 0)),
            pl.BlockSpec((1, h_dim), lambda i: (0, 0)),
        ],
        out_specs=[
            pl.BlockSpec((rb, h_dim), lambda i: (i, 0)),
            pl.BlockSpec((rb, 1), lambda i: (i, 0)),
        ],
        out_shape=[
            jax.ShapeDtypeStruct((n_pad, h_dim), jnp.float32),
            jax.ShapeDtypeStruct((n_pad, 1), jnp.float32),
        ],
    )(p1, p1, xp, W1l, W1r, b1.reshape(1, -1))

    seg2, n_pad2 = _make_sc_segsum(n, e, h_dim)
    assert n_pad2 == n_pad
    zeros2 = jnp.zeros((n_pad // NS, h_dim), jnp.float32)
    (p2,) = seg2(h, pk, zeros2)

    out = pl.pallas_call(
        _layer2_body,
        grid=(grid,),
        in_specs=[
            pl.BlockSpec((rb, h_dim), lambda i: (i, 0)),
            pl.BlockSpec((rb, h_dim), lambda i: (i + 8, 0)),
            pl.BlockSpec((rb, h_dim), lambda i: (i, 0)),
            pl.BlockSpec((rb, 1), lambda i: (i, 0)),
            pl.BlockSpec((h_dim, h_dim), lambda i: (0, 0)),
            pl.BlockSpec((h_dim, h_dim), lambda i: (0, 0)),
            pl.BlockSpec((1, h_dim), lambda i: (0, 0)),
            pl.BlockSpec((1, h_dim), lambda i: (0, 0)),
            pl.BlockSpec((1, 1), lambda i: (0, 0)),
        ],
        out_specs=pl.BlockSpec((rb, 1), lambda i: (i, 0)),
        out_shape=jax.ShapeDtypeStruct((n_pad, 1), jnp.float32),
    )(p2, p2, h, rcp, W2l, W2r, b2.reshape(1, -1), Wlin,
      blin.reshape(1, 1))
    return out[:n]


# speedup vs baseline: 1.0548x; 1.0548x over previous
import functools

import jax
import jax.numpy as jnp
import numpy as np
from jax import lax
from jax.experimental import pallas as pl
from jax.experimental.pallas import tpu as pltpu
from jax.experimental.pallas import tpu_sc as plsc

NC = 2
NS = 16
CHUNK = 80
PACK = 16384


def _make_sc_segsum(n, e, w):
    nw = NC * NS
    per_w = e // nw
    assert per_w * nw == e and per_w % CHUNK == 0
    nch = per_w // CHUNK
    assert nch >= 8 and nch % 3 == 2
    rpt = -(-n // (8 * NS)) * 8
    n_pad = rpt * NS
    mesh = plsc.VectorSubcoreMesh(core_axis_name="c", subcore_axis_name="s")

    split = w > 128

    def body(table, pk, zeros, *rest):
        if split:
            (out, outc, pk_v, sbuf0, sbuf1, sbuf2, dbuf0, dbuf1, dbuf2,
             rows0, rows1, rows2, acc, gsem, ssem) = rest
        else:
            (out, pk_v, sbuf0, sbuf1, sbuf2, dbuf0, dbuf1, dbuf2,
             rows0, rows1, rows2, acc, gsem, ssem) = rest
        c = lax.axis_index("c")
        s = lax.axis_index("s")
        g = s * NC + c
        pltpu.sync_copy(zeros, acc.at[pl.ds(s * rpt, rpt)])
        pltpu.sync_copy(pk.at[pl.ds(g * per_w, per_w)], pk_v)
        plsc.subcore_barrier()

        dummy = table.at[pl.ds(0, CHUNK)]
        sbufs = (sbuf0, sbuf1, sbuf2)
        dbufs = (dbuf0, dbuf1, dbuf2)
        rows = (rows0, rows1, rows2)

        def unpack(j, slot):
            base = j * CHUNK
            for k in range(CHUNK // 16):
                p16 = pk_v[pl.ds(base + k * 16, 16)]
                sbufs[slot][pl.ds(k * 16, 16)] = lax.bitwise_and(
                    p16, PACK - 1)
                dbufs[slot][pl.ds(k * 16, 16)] = lax.shift_right_logical(
                    p16, 14)

        def gather(slot):
            pltpu.async_copy(table.at[sbufs[slot]], rows[slot], gsem)

        def scatter(slot):
            pltpu.async_copy(rows[slot], acc.at[dbufs[slot]], ssem,
                             add=True)

        def drain(slot, sem):
            pltpu.make_async_copy(dummy, rows[slot], sem).wait()

        unpack(0, 0)
        gather(0)
        unpack(1, 1)
        gather(1)
        drain(0, gsem)
        scatter(0)
        unpack(2, 2)
        gather(2)
        drain(1, gsem)
        scatter(1)

        def triple(t, carry):
            for k in range(3):
                j = 3 * t + k
                drain(k, ssem)
                unpack(j, k)
                gather(k)
                drain((k + 2) % 3, gsem)
                scatter((k + 2) % 3)
            return carry

        lax.fori_loop(1, (nch - 2) // 3, triple, 0)
        drain(0, ssem)
        unpack(nch - 2, 0)
        gather(0)
        drain(2, gsem)
        scatter(2)
        drain(1, ssem)
        unpack(nch - 1, 1)
        gather(1)
        drain(0, gsem)
        scatter(0)
        drain(1, gsem)
        scatter(1)
        drain(2, ssem)
        drain(0, ssem)
        drain(1, ssem)
        plsc.subcore_barrier()
        if split:
            pltpu.sync_copy(acc.at[pl.ds(s * rpt, rpt), pl.ds(0, 128)],
                            out.at[pl.ds(c * n_pad + s * rpt, rpt)])
            pltpu.sync_copy(acc.at[pl.ds(s * rpt, rpt), pl.ds(128, w - 128)],
                            outc.at[pl.ds(c * n_pad + s * rpt, rpt)])
        else:
            pltpu.sync_copy(acc.at[pl.ds(s * rpt, rpt)],
                            out.at[pl.ds(c * n_pad + s * rpt, rpt)])

    if split:
        out_type = [
            jax.ShapeDtypeStruct((NC * n_pad, 128), jnp.float32),
            jax.ShapeDtypeStruct((NC * n_pad, w - 128), jnp.float32),
        ]
    else:
        out_type = [jax.ShapeDtypeStruct((NC * n_pad, w), jnp.float32)]
    scratch = (
        [pltpu.VMEM((per_w,), jnp.int32)]
        + [pltpu.VMEM((CHUNK,), jnp.int32)] * 6
        + [pltpu.VMEM((CHUNK, w), jnp.float32)] * 3
        + [
            pltpu.VMEM_SHARED((n_pad, w), jnp.float32),
            pltpu.SemaphoreType.DMA,
            pltpu.SemaphoreType.DMA,
        ]
    )
    return pl.kernel(
        body, out_type=out_type, mesh=mesh, scratch_types=scratch,
        compiler_params=pltpu.CompilerParams(
            needs_layout_passes=False, use_tc_tiling_on_sc=False),
    ), n_pad


def _dotT(a, b):
    return lax.dot_general(a, b, (((1,), (1,)), ((), ())),
                           preferred_element_type=jnp.float32)


def _layer1_body(pa_ref, pb_ref, ca_ref, cb_ref, xp_ref, wl_ref, wr_ref,
                 b_ref, h_ref, rcp_ref):
    d = wl_ref.shape[1]
    cnt = ca_ref[:, :1] + cb_ref[:, :1]
    rcp = 1.0 / jnp.maximum(cnt, 1.0)
    mean = (pa_ref[...] + pb_ref[...]) * rcp
    h = (_dotT(mean, wl_ref[...]) + _dotT(xp_ref[:, :d], wr_ref[...])
         + b_ref[...])
    h_ref[...] = jnp.maximum(h, 0.0)
    rcp_ref[...] = rcp


def _layer2_body(pa_ref, pb_ref, h_ref, rcp_ref, wl_ref, wr_ref, b_ref,
                 wlin_ref, blin_ref, o_ref):
    mean = (pa_ref[...] + pb_ref[...]) * rcp_ref[...]
    z = _dotT(mean, wl_ref[...]) + _dotT(h_ref[...], wr_ref[...]) + b_ref[...]
    z = jnp.maximum(z, 0.0)
    o_ref[...] = (jnp.sum(z * wlin_ref[...], axis=1, keepdims=True)
                  + blin_ref[0, 0])


def kernel(x, edge_index, W1l, W1r, b1, W2l, W2r, b2, Wlin, blin):
    n, d = x.shape
    e = edge_index.shape[1]
    h_dim = W1l.shape[0]
    per_w = e // (NC * NS)
    pk = edge_index[0] + edge_index[1] * PACK

    w1 = d + 8
    seg1, n_pad = _make_sc_segsum(n, e, w1)
    xp = jnp.zeros((n_pad, w1), jnp.float32)
    xp = lax.dynamic_update_slice(
        xp,
        jnp.concatenate([x, jnp.ones((n, 1), jnp.float32)], axis=1),
        (0, 0))

    zeros1 = jnp.zeros((n_pad // NS, w1), jnp.float32)
    p1, c1 = seg1(xp, pk, zeros1)

    rb = n_pad // 8
    assert rb % 8 == 0
    grid = n_pad // rb
    h, rcp = pl.pallas_call(
        _layer1_body,
        grid=(grid,),
        in_specs=[
            pl.BlockSpec((rb, h_dim), lambda i: (i, 0)),
            pl.BlockSpec((rb, h_dim), lambda i: (i + 8, 0)),
            pl.BlockSpec((rb, 8), lambda i: (i, 0)),
            pl.BlockSpec((rb, 8), lambda i: (i + 8, 0)),
            pl.BlockSpec((rb, w1), lambda i: (i, 0)),
            pl.BlockSpec((h_dim, d), lambda i: (0, 0)),
            pl.BlockSpec((h_dim, d), lambda i: (0, 0)),
            pl.BlockSpec((1, h_dim), lambda i: (0, 0)),
        ],
        out_specs=[
            pl.BlockSpec((rb, h_dim), lambda i: (i, 0)),
            pl.BlockSpec((rb, 1), lambda i: (i, 0)),
        ],
        out_shape=[
            jax.ShapeDtypeStruct((n_pad, h_dim), jnp.float32),
            jax.ShapeDtypeStruct((n_pad, 1), jnp.float32),
        ],
    )(p1, p1, c1, c1, xp, W1l, W1r, b1.reshape(1, -1))

    seg2, n_pad2 = _make_sc_segsum(n, e, h_dim)
    assert n_pad2 == n_pad
    zeros2 = jnp.zeros((n_pad // NS, h_dim), jnp.float32)
    (p2,) = seg2(h, pk, zeros2)

    out = pl.pallas_call(
        _layer2_body,
        grid=(grid,),
        in_specs=[
            pl.BlockSpec((rb, h_dim), lambda i: (i, 0)),
            pl.BlockSpec((rb, h_dim), lambda i: (i + 8, 0)),
            pl.BlockSpec((rb, h_dim), lambda i: (i, 0)),
            pl.BlockSpec((rb, 1), lambda i: (i, 0)),
            pl.BlockSpec((h_dim, h_dim), lambda i: (0, 0)),
            pl.BlockSpec((h_dim, h_dim), lambda i: (0, 0)),
            pl.BlockSpec((1, h_dim), lambda i: (0, 0)),
            pl.BlockSpec((1, h_dim), lambda i: (0, 0)),
            pl.BlockSpec((1, 1), lambda i: (0, 0)),
        ],
        out_specs=pl.BlockSpec((rb, 1), lambda i: (i, 0)),
        out_shape=jax.ShapeDtypeStruct((n_pad, 1), jnp.float32),
    )(p2, p2, h, rcp, W2l, W2r, b2.reshape(1, -1), Wlin,
      blin.reshape(1, 1))
    return out[:n]


# speedup vs baseline: 1.0951x; 1.0382x over previous
import functools

import jax
import jax.numpy as jnp
import numpy as np
from jax import lax
from jax.experimental import pallas as pl
from jax.experimental.pallas import tpu as pltpu
from jax.experimental.pallas import tpu_sc as plsc

NC = 2
NS = 16
CHUNK = 80
PACK = 16384


def _make_sc_segsum(n, e, w):
    nw = NC * NS
    per_w = e // nw
    assert per_w * nw == e and per_w % CHUNK == 0
    nch = per_w // CHUNK
    assert nch >= 8 and nch % 3 == 2
    rpt = -(-n // (8 * NS)) * 8
    n_pad = rpt * NS
    mesh = plsc.VectorSubcoreMesh(core_axis_name="c", subcore_axis_name="s")

    split = w > 128

    def body(table, pk, zeros, *rest):
        if split:
            (out, outc, pk_v, sbuf0, sbuf1, sbuf2, dbuf0, dbuf1, dbuf2,
             rows0, rows1, rows2, acc, gsem, ssem) = rest
        else:
            (out, pk_v, sbuf0, sbuf1, sbuf2, dbuf0, dbuf1, dbuf2,
             rows0, rows1, rows2, acc, gsem, ssem) = rest
        c = lax.axis_index("c")
        s = lax.axis_index("s")
        g = s * NC + c
        pltpu.sync_copy(zeros, acc.at[pl.ds(s * rpt, rpt)])
        pltpu.sync_copy(pk.at[pl.ds(g * per_w, per_w)], pk_v)
        plsc.subcore_barrier()

        dummy = table.at[pl.ds(0, CHUNK)]
        sbufs = (sbuf0, sbuf1, sbuf2)
        dbufs = (dbuf0, dbuf1, dbuf2)
        rows = (rows0, rows1, rows2)

        def unpack(j, slot):
            base = j * CHUNK
            for k in range(CHUNK // 16):
                p16 = pk_v[pl.ds(base + k * 16, 16)]
                sbufs[slot][pl.ds(k * 16, 16)] = lax.bitwise_and(
                    p16, PACK - 1)
                dbufs[slot][pl.ds(k * 16, 16)] = lax.shift_right_logical(
                    p16, 14)

        def gather(slot):
            pltpu.async_copy(table.at[sbufs[slot]], rows[slot], gsem)

        def scatter(slot):
            pltpu.async_copy(rows[slot], acc.at[dbufs[slot]], ssem,
                             add=True)

        def drain(slot, sem):
            pltpu.make_async_copy(dummy, rows[slot], sem).wait()

        unpack(0, 0)
        gather(0)
        unpack(1, 1)
        gather(1)
        drain(0, gsem)
        scatter(0)
        unpack(2, 2)
        gather(2)
        drain(1, gsem)
        scatter(1)

        def triple(t, carry):
            for k in range(3):
                j = 3 * t + k
                drain(k, ssem)
                unpack(j, k)
                gather(k)
                drain((k + 2) % 3, gsem)
                scatter((k + 2) % 3)
            return carry

        lax.fori_loop(1, (nch - 2) // 3, triple, 0)
        drain(0, ssem)
        unpack(nch - 2, 0)
        gather(0)
        drain(2, gsem)
        scatter(2)
        drain(1, ssem)
        unpack(nch - 1, 1)
        gather(1)
        drain(0, gsem)
        scatter(0)
        drain(1, gsem)
        scatter(1)
        drain(2, ssem)
        drain(0, ssem)
        drain(1, ssem)
        plsc.subcore_barrier()
        if split:
            pltpu.sync_copy(acc.at[pl.ds(s * rpt, rpt), pl.ds(0, 128)],
                            out.at[pl.ds(c * n_pad + s * rpt, rpt)])
            pltpu.sync_copy(acc.at[pl.ds(s * rpt, rpt), pl.ds(128, w - 128)],
                            outc.at[pl.ds(c * n_pad + s * rpt, rpt)])
        else:
            pltpu.sync_copy(acc.at[pl.ds(s * rpt, rpt)],
                            out.at[pl.ds(c * n_pad + s * rpt, rpt)])

    if split:
        out_type = [
            jax.ShapeDtypeStruct((NC * n_pad, 128), jnp.float32),
            jax.ShapeDtypeStruct((NC * n_pad, w - 128), jnp.float32),
        ]
    else:
        out_type = [jax.ShapeDtypeStruct((NC * n_pad, w), jnp.float32)]
    scratch = (
        [pltpu.VMEM((per_w,), jnp.int32)]
        + [pltpu.VMEM((CHUNK,), jnp.int32)] * 6
        + [pltpu.VMEM((CHUNK, w), jnp.float32)] * 3
        + [
            pltpu.VMEM_SHARED((n_pad, w), jnp.float32),
            pltpu.SemaphoreType.DMA,
            pltpu.SemaphoreType.DMA,
        ]
    )
    return pl.kernel(
        body, out_type=out_type, mesh=mesh, scratch_types=scratch,
        compiler_params=pltpu.CompilerParams(
            needs_layout_passes=False, use_tc_tiling_on_sc=False),
    ), n_pad


def _make_sc_pack(e):
    nw = NC * NS
    per_w = e // nw
    assert per_w * nw == e and per_w % 16 == 0
    mesh = plsc.VectorSubcoreMesh(core_axis_name="c", subcore_axis_name="s")

    def body(ei, out, sv, dv, pv):
        c = lax.axis_index("c")
        s = lax.axis_index("s")
        g = s * NC + c
        pltpu.sync_copy(ei.at[0, pl.ds(g * per_w, per_w)], sv)
        pltpu.sync_copy(ei.at[1, pl.ds(g * per_w, per_w)], dv)

        def step(i, carry):
            o = i * 16
            pv[pl.ds(o, 16)] = lax.bitwise_or(
                sv[pl.ds(o, 16)],
                lax.shift_left(dv[pl.ds(o, 16)], 14))
            return carry

        lax.fori_loop(0, per_w // 16, step, 0)
        pltpu.sync_copy(pv, out.at[pl.ds(g * per_w, per_w)])

    return pl.kernel(
        body,
        out_type=[jax.ShapeDtypeStruct((e,), jnp.int32)],
        mesh=mesh,
        scratch_types=[pltpu.VMEM((per_w,), jnp.int32)] * 3,
        compiler_params=pltpu.CompilerParams(
            needs_layout_passes=False, use_tc_tiling_on_sc=False),
    )


def _dotT(a, b):
    return lax.dot_general(a, b, (((1,), (1,)), ((), ())),
                           preferred_element_type=jnp.float32)


def _layer1_body(pa_ref, pb_ref, ca_ref, cb_ref, xp_ref, wl_ref, wr_ref,
                 b_ref, h_ref, rcp_ref):
    d = wl_ref.shape[1]
    cnt = ca_ref[:, :1] + cb_ref[:, :1]
    rcp = 1.0 / jnp.maximum(cnt, 1.0)
    mean = (pa_ref[...] + pb_ref[...]) * rcp
    h = (_dotT(mean, wl_ref[...]) + _dotT(xp_ref[:, :d], wr_ref[...])
         + b_ref[...])
    h_ref[...] = jnp.maximum(h, 0.0)
    rcp_ref[...] = rcp


def _layer2_body(pa_ref, pb_ref, h_ref, rcp_ref, wl_ref, wr_ref, b_ref,
                 wlin_ref, blin_ref, o_ref):
    mean = (pa_ref[...] + pb_ref[...]) * rcp_ref[...]
    z = _dotT(mean, wl_ref[...]) + _dotT(h_ref[...], wr_ref[...]) + b_ref[...]
    z = jnp.maximum(z, 0.0)
    o_ref[...] = (jnp.sum(z * wlin_ref[...], axis=1, keepdims=True)
                  + blin_ref[0, 0])


def kernel(x, edge_index, W1l, W1r, b1, W2l, W2r, b2, Wlin, blin):
    n, d = x.shape
    e = edge_index.shape[1]
    h_dim = W1l.shape[0]
    (pk,) = _make_sc_pack(e)(edge_index)

    w1 = d + 8
    seg1, n_pad = _make_sc_segsum(n, e, w1)
    xp = jnp.zeros((n_pad, w1), jnp.float32)
    xp = lax.dynamic_update_slice(
        xp,
        jnp.concatenate([x, jnp.ones((n, 1), jnp.float32)], axis=1),
        (0, 0))

    zeros1 = jnp.zeros((n_pad // NS, w1), jnp.float32)
    p1, c1 = seg1(xp, pk, zeros1)

    rb = n_pad // 8
    assert rb % 8 == 0
    grid = n_pad // rb
    h, rcp = pl.pallas_call(
        _layer1_body,
        grid=(grid,),
        in_specs=[
            pl.BlockSpec((rb, h_dim), lambda i: (i, 0)),
            pl.BlockSpec((rb, h_dim), lambda i: (i + 8, 0)),
            pl.BlockSpec((rb, 8), lambda i: (i, 0)),
            pl.BlockSpec((rb, 8), lambda i: (i + 8, 0)),
            pl.BlockSpec((rb, w1), lambda i: (i, 0)),
            pl.BlockSpec((h_dim, d), lambda i: (0, 0)),
            pl.BlockSpec((h_dim, d), lambda i: (0, 0)),
            pl.BlockSpec((1, h_dim), lambda i: (0, 0)),
        ],
        out_specs=[
            pl.BlockSpec((rb, h_dim), lambda i: (i, 0)),
            pl.BlockSpec((rb, 1), lambda i: (i, 0)),
        ],
        out_shape=[
            jax.ShapeDtypeStruct((n_pad, h_dim), jnp.float32),
            jax.ShapeDtypeStruct((n_pad, 1), jnp.float32),
        ],
    )(p1, p1, c1, c1, xp, W1l, W1r, b1.reshape(1, -1))

    seg2, n_pad2 = _make_sc_segsum(n, e, h_dim)
    assert n_pad2 == n_pad
    zeros2 = jnp.zeros((n_pad // NS, h_dim), jnp.float32)
    (p2,) = seg2(h, pk, zeros2)

    out = pl.pallas_call(
        _layer2_body,
        grid=(grid,),
        in_specs=[
            pl.BlockSpec((rb, h_dim), lambda i: (i, 0)),
            pl.BlockSpec((rb, h_dim), lambda i: (i + 8, 0)),
            pl.BlockSpec((rb, h_dim), lambda i: (i, 0)),
            pl.BlockSpec((rb, 1), lambda i: (i, 0)),
            pl.BlockSpec((h_dim, h_dim), lambda i: (0, 0)),
            pl.BlockSpec((h_dim, h_dim), lambda i: (0, 0)),
            pl.BlockSpec((1, h_dim), lambda i: (0, 0)),
            pl.BlockSpec((1, h_dim), lambda i: (0, 0)),
            pl.BlockSpec((1, 1), lambda i: (0, 0)),
        ],
        out_specs=pl.BlockSpec((rb, 1), lambda i: (i, 0)),
        out_shape=jax.ShapeDtypeStruct((n_pad, 1), jnp.float32),
    )(p2, p2, h, rcp, W2l, W2r, b2.reshape(1, -1), Wlin,
      blin.reshape(1, 1))
    return out[:n]
